# baseline (speedup 1.0000x reference)
import functools

import jax
import jax.numpy as jnp
from jax import lax
from jax.experimental import pallas as pl
from jax.experimental.pallas import tpu as pltpu
from jax.experimental.pallas import tpu_sc as plsc

_MAX_REL = 128
_HEAD = 64
_VOCAB = 2 * _MAX_REL + 1
_SEQ = 2048
_NPHASE = 128
_PER_PHASE = _SEQ // _NPHASE
_NW = 32
_BPW = 160
_EXT = _NW * _BPW
_CHUNKS = (128, 32)


def _sc_build_r(emb_hbm, r_hbm, idx_a, idx_b, rows_v, sem):
    wid = lax.axis_index("s") * 2 + lax.axis_index("c")
    m0 = wid * _BPW
    for base, ref in ((0, idx_a), (_CHUNKS[0], idx_b)):
        n = ref.shape[0]
        for c in range(n // 16):
            m = m0 + base + c * 16 + lax.iota(jnp.int32, 16)
            ref[pl.ds(c * 16, 16)] = jnp.clip(2175 - m, 0, _VOCAB - 1)
    pltpu.async_copy(emb_hbm.at[idx_a], rows_v.at[pl.ds(0, _CHUNKS[0])],
                     sem).wait()
    pltpu.async_copy(emb_hbm.at[idx_b],
                     rows_v.at[pl.ds(_CHUNKS[0], _CHUNKS[1])], sem).wait()
    pltpu.sync_copy(rows_v, r_hbm.at[pl.ds(m0, _BPW)])


def _tc_expand(r_ref, out_ref, rt, shifted):
    b = pl.program_id(0)

    @pl.when(b == 0)
    def _():
        rt[:] = r_ref[:, :_HEAD].T

    phase = (_NPHASE - 1) - b
    shifted[:] = pltpu.roll(rt[:], _EXT - phase, 1)
    for a in range(_PER_PHASE):
        base = (_PER_PHASE - 1 - a) * _NPHASE
        out_ref[a, 0] = shifted[:, base:base + _SEQ]


def kernel(seq_len, embedding):
    del seq_len

    sc_build = functools.partial(
        pl.kernel,
        out_type=jax.ShapeDtypeStruct((_EXT, 2 * _HEAD), jnp.float32),
        mesh=plsc.VectorSubcoreMesh(core_axis_name="c", subcore_axis_name="s"),
        scratch_types=[
            pltpu.VMEM((_CHUNKS[0],), jnp.int32),
            pltpu.VMEM((_CHUNKS[1],), jnp.int32),
            pltpu.VMEM((_BPW, 2 * _HEAD), jnp.float32),
            pltpu.SemaphoreType.DMA,
        ],
    )(_sc_build_r)
    emb_pad = jnp.zeros((_VOCAB, 2 * _HEAD), jnp.float32).at[:, :_HEAD].set(embedding)
    r = sc_build(emb_pad)

    out4 = pl.pallas_call(
        _tc_expand,
        grid=(_NPHASE,),
        in_specs=[pl.BlockSpec((_EXT, 2 * _HEAD), lambda b: (0, 0))],
        out_specs=pl.BlockSpec((_PER_PHASE, 1, _HEAD, _SEQ),
                               lambda b: (0, b, 0, 0)),
        out_shape=jax.ShapeDtypeStruct((_PER_PHASE, _NPHASE, _HEAD, _SEQ),
                                       jnp.float32),
        scratch_shapes=[pltpu.VMEM((_HEAD, _EXT), jnp.float32),
                        pltpu.VMEM((_HEAD, _EXT), jnp.float32)],
    )(r)
    out_t = out4.reshape(_SEQ, _HEAD, _SEQ)
    return jnp.transpose(out_t, (0, 2, 1))


# speedup vs baseline: 1.1024x; 1.1024x over previous
import functools

import jax
import jax.numpy as jnp
from jax import lax
from jax.experimental import pallas as pl
from jax.experimental.pallas import tpu as pltpu
from jax.experimental.pallas import tpu_sc as plsc

_MAX_REL = 128
_HEAD = 64
_VOCAB = 2 * _MAX_REL + 1
_SEQ = 2048
_NPHASE = 128
_PER_PHASE = _SEQ // _NPHASE
_NW = 32
_BPW = 128
_EXT = _NW * _BPW


def _sc_build_r(emb_hbm, r_hbm, idx_v, rows_v, sem):
    wid = lax.axis_index("s") * 2 + lax.axis_index("c")
    m0 = wid * _BPW
    for c in range(_BPW // 16):
        m = m0 + c * 16 + lax.iota(jnp.int32, 16)
        idx_v[pl.ds(c * 16, 16)] = jnp.clip(2175 - m, 0, _VOCAB - 1)
    pltpu.async_copy(emb_hbm.at[idx_v], rows_v, sem).wait()
    pltpu.sync_copy(rows_v, r_hbm.at[pl.ds(m0, _BPW)])


def _tc_expand(r_ref, out_ref, rt, shifted):
    b = pl.program_id(0)

    @pl.when(b == 0)
    def _():
        rt[:] = r_ref[:, :_HEAD].T

    phase = (_NPHASE - 1) - b
    shifted[:] = pltpu.roll(rt[:], _EXT - phase, 1)
    for a in range(_PER_PHASE):
        base = (_PER_PHASE - 1 - a) * _NPHASE
        out_ref[a, 0] = shifted[:, base:base + _SEQ]


def kernel(seq_len, embedding):
    del seq_len

    sc_build = functools.partial(
        pl.kernel,
        out_type=jax.ShapeDtypeStruct((_EXT, 2 * _HEAD), jnp.float32),
        mesh=plsc.VectorSubcoreMesh(core_axis_name="c", subcore_axis_name="s"),
        scratch_types=[
            pltpu.VMEM((_BPW,), jnp.int32),
            pltpu.VMEM((_BPW, 2 * _HEAD), jnp.float32),
            pltpu.SemaphoreType.DMA,
        ],
    )(_sc_build_r)
    emb_pad = jnp.zeros((_VOCAB, 2 * _HEAD), jnp.float32).at[:, :_HEAD].set(embedding)
    r = sc_build(emb_pad)

    out4 = pl.pallas_call(
        _tc_expand,
        grid=(_NPHASE,),
        in_specs=[pl.BlockSpec((_EXT, 2 * _HEAD), lambda b: (0, 0))],
        out_specs=pl.BlockSpec((_PER_PHASE, 1, _HEAD, _SEQ),
                               lambda b: (0, b, 0, 0)),
        out_shape=jax.ShapeDtypeStruct((_PER_PHASE, _NPHASE, _HEAD, _SEQ),
                                       jnp.float32),
        scratch_shapes=[pltpu.VMEM((_HEAD, _EXT), jnp.float32),
                        pltpu.VMEM((_HEAD, _EXT), jnp.float32)],
    )(r)
    out_t = out4.reshape(_SEQ, _HEAD, _SEQ)
    return jnp.transpose(out_t, (0, 2, 1))
